# baseline (speedup 1.0000x reference)
import functools

import jax
import jax.numpy as jnp
from jax import lax
from jax.experimental import pallas as pl
from jax.experimental.pallas import tpu as pltpu
from jax.experimental.pallas import tpu_sc as plsc

_P = 64
_A = 16
_NC = 2
_NS = 16
_NW = _NC * _NS
_RPW = _P // _NW
_L = 16
_CHUNK = 5120
_TB = 512


def _sc_gather(W, tok):
    V = W.shape[1]
    N = tok.shape[0]
    mesh = plsc.VectorSubcoreMesh(
        core_axis_name="c", subcore_axis_name="s",
        num_cores=_NC, num_subcores=_NS)

    @functools.partial(
        pl.kernel,
        out_type=jax.ShapeDtypeStruct((_P, N), jnp.float32),
        mesh=mesh,
        compiler_params=pltpu.CompilerParams(needs_layout_passes=False),
        scratch_types=[
            pltpu.VMEM((N,), jnp.int32),
            pltpu.VMEM((V,), jnp.float32),
            pltpu.VMEM((_CHUNK,), jnp.float32)
        ],
    )
    def k(w_hbm, tok_hbm, ut_hbm, tok_v, w_v, out_v):
        wid = lax.axis_index("s") * _NC + lax.axis_index("c")
        pltpu.sync_copy(tok_hbm, tok_v)
        for r in range(_RPW):
            p = wid * _RPW + r
            pltpu.sync_copy(w_hbm.at[p], w_v)

            def chunk_body(c, _, p=p):
                base = pl.multiple_of(c * _CHUNK, _CHUNK)

                def g(i, _):
                    idx = tok_v[pl.ds(base + i * _L, _L)]
                    out_v[pl.ds(i * _L, _L)] = plsc.load_gather(w_v, [idx])
                    return 0

                lax.fori_loop(0, _CHUNK // _L, g, 0, unroll=8)
                pltpu.sync_copy(out_v, ut_hbm.at[p, pl.ds(base, _CHUNK)])
                return 0

            lax.fori_loop(0, N // _CHUNK, chunk_body, 0)

    return k(W, tok)


def _tc_expand(UT, SPc, fT):
    N = UT.shape[1]
    PA = _P * _A

    def body(ut_ref, sp_ref, ft_ref, fh_ref, y_ref):
        ut = ut_ref[...]
        fh = (ut >= sp_ref[...]).astype(jnp.float32)
        t = fh * ut
        ftv = ft_ref[...]
        tiled = jnp.concatenate([ftv] * _P, axis=0)
        qq = lax.broadcasted_iota(jnp.int32, (PA, _P), 0) // _A
        pp = lax.broadcasted_iota(jnp.int32, (PA, _P), 1)
        et = jnp.where(qq == pp, tiled, 0.0)
        y = jnp.dot(et, t, preferred_element_type=jnp.float32)
        fh_ref[...] = fh
        y_ref[...] = y

    return pl.pallas_call(
        body,
        grid=(N // _TB,),
        in_specs=[
            pl.BlockSpec((_P, _TB), lambda j: (0, j)),
            pl.BlockSpec((_P, 1), lambda j: (0, 0)),
            pl.BlockSpec((_A, _P), lambda j: (0, 0)),
        ],
        out_specs=[
            pl.BlockSpec((_P, _TB), lambda j: (0, j)),
            pl.BlockSpec((PA, _TB), lambda j: (0, j)),
        ],
        out_shape=[
            jax.ShapeDtypeStruct((_P, N), jnp.float32),
            jax.ShapeDtypeStruct((PA, N), jnp.float32),
        ],
    )(UT, SPc, fT)


def kernel(input_ids, W, f, SP):
    B, T = input_ids.shape
    N = B * T
    tok = input_ids.reshape(N)
    UT = _sc_gather(W, tok)
    FhT, Y2T = _tc_expand(UT, SP.reshape(_P, 1), f.T)
    U = UT.T
    Fh = FhT.T
    Y = Y2T.reshape(_P, _A, N).transpose(2, 0, 1)
    return U, Fh, Y


# speedup vs baseline: 1.1461x; 1.1461x over previous
import functools

import jax
import jax.numpy as jnp
from jax import lax
from jax.experimental import pallas as pl
from jax.experimental.pallas import tpu as pltpu
from jax.experimental.pallas import tpu_sc as plsc

_P = 64
_A = 16
_NC = 2
_NS = 16
_NW = _NC * _NS
_RPW = _P // _NW
_L = 16
_CHUNK = 5120
_TB = 512


def _sc_gather(W, tok):
    V = W.shape[1]
    N = tok.shape[0]
    mesh = plsc.VectorSubcoreMesh(
        core_axis_name="c", subcore_axis_name="s",
        num_cores=_NC, num_subcores=_NS)

    @functools.partial(
        pl.kernel,
        out_type=jax.ShapeDtypeStruct((_P, N), jnp.float32),
        mesh=mesh,
        compiler_params=pltpu.CompilerParams(needs_layout_passes=False),
        scratch_types=[
            pltpu.VMEM((N,), jnp.int32),
            pltpu.VMEM((V,), jnp.float32),
            pltpu.VMEM((_CHUNK,), jnp.float32)
        ],
    )
    def k(w_hbm, tok_hbm, ut_hbm, tok_v, w_v, out_v):
        wid = lax.axis_index("s") * _NC + lax.axis_index("c")
        pltpu.sync_copy(tok_hbm, tok_v)
        for r in range(_RPW):
            p = wid * _RPW + r
            pltpu.sync_copy(w_hbm.at[p], w_v)

            def chunk_body(c, _, p=p):
                base = pl.multiple_of(c * _CHUNK, _CHUNK)

                def g(i, _):
                    idx = tok_v[pl.ds(base + i * _L, _L)]
                    out_v[pl.ds(i * _L, _L)] = plsc.load_gather(w_v, [idx])
                    return 0

                lax.fori_loop(0, _CHUNK // _L, g, 0, unroll=8)
                pltpu.sync_copy(out_v, ut_hbm.at[p, pl.ds(base, _CHUNK)])
                return 0

            lax.fori_loop(0, N // _CHUNK, chunk_body, 0)

    return k(W, tok)


def _tc_expand(UT, SPc, fT):
    N = UT.shape[1]
    PA = _P * _A

    QR = 128

    def body(ut_ref, sp_ref, ft_ref, fh_ref, y_ref):
        q = pl.program_id(0)
        ut = ut_ref[...]
        fh = (ut >= sp_ref[...]).astype(jnp.float32)
        t = fh * ut
        ftv = ft_ref[...]
        tiled = jnp.concatenate([ftv] * (QR // _A), axis=0)
        rr = lax.broadcasted_iota(jnp.int32, (QR, _P), 0) + q * QR
        pp = lax.broadcasted_iota(jnp.int32, (QR, _P), 1)
        et_q = jnp.where(rr // _A == pp, tiled, 0.0)
        y_ref[...] = jnp.dot(et_q, t, preferred_element_type=jnp.float32)
        fh_ref[...] = fh

    return pl.pallas_call(
        body,
        grid=(PA // QR,),
        in_specs=[
            pl.BlockSpec((_P, N), lambda q: (0, 0)),
            pl.BlockSpec((_P, 1), lambda q: (0, 0)),
            pl.BlockSpec((_A, _P), lambda q: (0, 0)),
        ],
        out_specs=[
            pl.BlockSpec((_P, N), lambda q: (0, 0)),
            pl.BlockSpec((QR, N), lambda q: (q, 0)),
        ],
        out_shape=[
            jax.ShapeDtypeStruct((_P, N), jnp.float32),
            jax.ShapeDtypeStruct((PA, N), jnp.float32),
        ],
    )(UT, SPc, fT)


def kernel(input_ids, W, f, SP):
    B, T = input_ids.shape
    N = B * T
    tok = input_ids.reshape(N)
    UT = _sc_gather(W, tok)
    FhT, Y2T = _tc_expand(UT, SP.reshape(_P, 1), f.T)
    U = UT.T
    Fh = FhT.T
    Y = Y2T.reshape(_P, _A, N).transpose(2, 0, 1)
    return U, Fh, Y
